# Initial kernel scaffold; baseline (speedup 1.0000x reference)
import jax
import jax.numpy as jnp
from jax.experimental import pallas as pl


def kernel(x, edge_index, W):
    raise NotImplementedError("write your pallas kernel here")


# speedup vs baseline: 2.9682x; 2.9682x over previous
import functools

import jax
import jax.numpy as jnp
from jax import lax
from jax.experimental import pallas as pl
from jax.experimental.pallas import tpu as pltpu
from jax.experimental.pallas import tpu_sc as plsc

_LANES = 16
_NC = 2
_NS = 16
_NW = _NC * _NS


def _mm_body(x_ref, w_ref, y_ref):
    y_ref[...] = lax.dot_general(
        x_ref[...], w_ref[...],
        dimension_numbers=(((1,), (1,)), ((), ())),
        preferred_element_type=jnp.float32,
    )


def _dense_stage(x, W):
    n, _ = x.shape
    o = W.shape[0]
    return pl.pallas_call(
        _mm_body,
        out_shape=jax.ShapeDtypeStruct((n, o), jnp.float32),
    )(x, W)


def _make_sc_stage(n, e, o):
    ew = e // _NW
    groups = ew // _LANES
    mesh = plsc.VectorSubcoreMesh(core_axis_name="c", subcore_axis_name="s")

    @functools.partial(
        pl.kernel,
        mesh=mesh,
        compiler_params=pltpu.CompilerParams(needs_layout_passes=False),
        out_type=jax.ShapeDtypeStruct((e * o,), jnp.float32),
        scratch_types=[
            pltpu.VMEM((n * o,), jnp.float32),
            pltpu.VMEM((ew,), jnp.int32),
            pltpu.VMEM((ew,), jnp.int32),
            pltpu.VMEM((ew * o,), jnp.float32),
        ],
    )
    def sc_edge_tanh(y_hbm, row_hbm, col_hbm, out_hbm, y_v, row_v, col_v, out_v):
        wid = lax.axis_index("s") * _NC + lax.axis_index("c")
        base = wid * ew
        pltpu.sync_copy(y_hbm, y_v)
        pltpu.sync_copy(row_hbm.at[pl.ds(base, ew)], row_v)
        pltpu.sync_copy(col_hbm.at[pl.ds(base, ew)], col_v)
        lane = lax.iota(jnp.int32, _LANES)

        def step(i, carry):
            off = i * _LANES
            ridx = row_v[pl.ds(off, _LANES)] * o
            cidx = col_v[pl.ds(off, _LANES)] * o
            ob = off * o + lane * o
            for k in range(o):
                a = (plsc.load_gather(y_v, [ridx + k])
                     + plsc.load_gather(y_v, [cidx + k]))
                t = 1.0 - 2.0 / (jnp.exp(a + a) + 1.0)
                plsc.store_scatter(out_v, [ob + k], t)
            return carry

        lax.fori_loop(0, groups, step, 0)
        pltpu.sync_copy(out_v, out_hbm.at[pl.ds(base * o, ew * o)])

    return sc_edge_tanh


def kernel(x, edge_index, W):
    n = x.shape[0]
    o = W.shape[0]
    e = edge_index.shape[1]
    y = _dense_stage(x, W)
    out_flat = _make_sc_stage(n, e, o)(
        y.reshape(-1), edge_index[0], edge_index[1])
    return out_flat.reshape(e, 2, 2)


# speedup vs baseline: 28.8695x; 9.7263x over previous
import functools

import jax
import jax.numpy as jnp
from jax import lax
from jax.experimental import pallas as pl
from jax.experimental.pallas import tpu as pltpu
from jax.experimental.pallas import tpu_sc as plsc

_LANES = 16
_NC = 2
_NS = 16
_NW = _NC * _NS


def _mm_body(x_ref, w_ref, y_ref):
    y_ref[...] = lax.dot_general(
        x_ref[...], w_ref[...],
        dimension_numbers=(((1,), (1,)), ((), ())),
        preferred_element_type=jnp.float32,
    )


def _dense_stage(x, W):
    n, _ = x.shape
    o = W.shape[0]
    return pl.pallas_call(
        _mm_body,
        out_shape=jax.ShapeDtypeStruct((n, o), jnp.float32),
    )(x, W)


def _make_sc_stage(n, e, o):
    ew = e // _NW
    groups = ew // _LANES
    mesh = plsc.VectorSubcoreMesh(core_axis_name="c", subcore_axis_name="s")

    @functools.partial(
        pl.kernel,
        mesh=mesh,
        compiler_params=pltpu.CompilerParams(needs_layout_passes=False),
        out_type=jax.ShapeDtypeStruct((e * o,), jnp.float32),
        scratch_types=[
            pltpu.VMEM((n * o,), jnp.float32),
            pltpu.VMEM((ew,), jnp.int32),
            pltpu.VMEM((ew,), jnp.int32),
        ] + [pltpu.VMEM((ew,), jnp.float32) for _ in range(o)],
    )
    def sc_edge_tanh(y_hbm, row_hbm, col_hbm, out_hbm, y_v, row_v, col_v, *out_vs):
        wid = lax.axis_index("s") * _NC + lax.axis_index("c")
        base = wid * ew
        pltpu.sync_copy(y_hbm, y_v)
        pltpu.sync_copy(row_hbm.at[pl.ds(base, ew)], row_v)
        pltpu.sync_copy(col_hbm.at[pl.ds(base, ew)], col_v)

        def step(i, carry):
            off = i * _LANES
            ridx = row_v[pl.ds(off, _LANES)] * o
            cidx = col_v[pl.ds(off, _LANES)] * o
            for k in range(o):
                a = (plsc.load_gather(y_v, [ridx + k])
                     + plsc.load_gather(y_v, [cidx + k]))
                out_vs[k][pl.ds(off, _LANES)] = 1.0 - 2.0 / (jnp.exp(a + a) + 1.0)
            return carry

        lax.fori_loop(0, groups, step, 0)
        for k in range(o):
            pltpu.sync_copy(out_vs[k], out_hbm.at[pl.ds(k * e + base, ew)])

    return sc_edge_tanh


def kernel(x, edge_index, W):
    n = x.shape[0]
    o = W.shape[0]
    e = edge_index.shape[1]
    y = _dense_stage(x, W)
    out_t = _make_sc_stage(n, e, o)(
        y.reshape(-1), edge_index[0], edge_index[1])
    return out_t.reshape(2, 2, e).transpose(2, 0, 1)


# speedup vs baseline: 49.1204x; 1.7015x over previous
import functools

import jax
import jax.numpy as jnp
from jax import lax
from jax.experimental import pallas as pl
from jax.experimental.pallas import tpu as pltpu
from jax.experimental.pallas import tpu_sc as plsc

_LANES = 16
_NC = 2
_NS = 16
_NW = _NC * _NS


def _mm_body(x_ref, w_ref, y_ref):
    y_ref[...] = lax.dot_general(
        x_ref[...], w_ref[...],
        dimension_numbers=(((1,), (1,)), ((), ())),
        preferred_element_type=jnp.float32,
    )


def _dense_stage(x, W):
    n, _ = x.shape
    o = W.shape[0]
    return pl.pallas_call(
        _mm_body,
        out_shape=jax.ShapeDtypeStruct((n, o), jnp.float32),
    )(x, W)


def _make_sc_stage(n, e, o):
    ew = e // _NW
    groups = ew // _LANES
    mesh = plsc.VectorSubcoreMesh(core_axis_name="c", subcore_axis_name="s")

    @functools.partial(
        pl.kernel,
        mesh=mesh,
        compiler_params=pltpu.CompilerParams(needs_layout_passes=False),
        out_type=jax.ShapeDtypeStruct((e * o,), jnp.float32),
        scratch_types=[
            pltpu.VMEM((n * o,), jnp.float32),
            pltpu.VMEM((ew,), jnp.int32),
            pltpu.VMEM((ew,), jnp.int32),
        ] + [pltpu.VMEM((ew,), jnp.float32) for _ in range(o)],
    )
    def sc_edge_tanh(y_hbm, row_hbm, col_hbm, out_hbm, y_v, row_v, col_v, *out_vs):
        wid = lax.axis_index("s") * _NC + lax.axis_index("c")
        base = wid * ew
        pltpu.sync_copy(y_hbm, y_v)
        pltpu.sync_copy(row_hbm.at[pl.ds(base, ew)], row_v)
        pltpu.sync_copy(col_hbm.at[pl.ds(base, ew)], col_v)

        @plsc.parallel_loop(0, groups, unroll=8)
        def step(i):
            off = i * _LANES
            ridx = row_v[pl.ds(off, _LANES)] * o
            cidx = col_v[pl.ds(off, _LANES)] * o
            for k in range(o):
                a = (plsc.load_gather(y_v, [ridx + k])
                     + plsc.load_gather(y_v, [cidx + k]))
                out_vs[k][pl.ds(off, _LANES)] = 1.0 - 2.0 / (jnp.exp(a + a) + 1.0)
        for k in range(o):
            pltpu.sync_copy(out_vs[k], out_hbm.at[pl.ds(k * e + base, ew)])

    return sc_edge_tanh


def kernel(x, edge_index, W):
    n = x.shape[0]
    o = W.shape[0]
    e = edge_index.shape[1]
    y = _dense_stage(x, W)
    out_t = _make_sc_stage(n, e, o)(
        y.reshape(-1), edge_index[0], edge_index[1])
    return out_t.reshape(2, 2, e).transpose(2, 0, 1)
